# Initial kernel scaffold; baseline (speedup 1.0000x reference)
import jax
import jax.numpy as jnp
from jax.experimental import pallas as pl


def kernel(x_drug, x_disease, edge_index_dd, edge_index_dr, edge_label_index, Wl1_dd, bl1_dd, Wr1_dd, Wl1_dr, bl1_dr, Wr1_dr, Wl2_dd, bl2_dd, Wr2_dd, Wl2_dr, bl2_dr, Wr2_dr, W1, b1, W2, b2):
    raise NotImplementedError("write your pallas kernel here")


# speedup vs baseline: 3.3184x; 3.3184x over previous
import functools

import jax
import jax.numpy as jnp
from jax import lax
from jax.experimental import pallas as pl
from jax.experimental.pallas import tpu as pltpu
from jax.experimental.pallas import tpu_sc as plsc

N_NODE = 10000
NPAD = 10240
D = 128
E_EDGE = 320000
E_LBL = 100000

NSC = 2
NTILE = 16
CH = 128
K_CONV = 160
G_CONV = 16
EPT = NTILE * K_CONV * CH
STRIPE = NPAD // NTILE

VLBL = 102400
K_LBL = VLBL // (NSC * NTILE * CH)


def _zero_fill_2d(ref, rows):
    def row(r, c):
        for i in range(D // 16):
            ref[r, pl.ds(i * 16, 16)] = jnp.zeros((16,), jnp.float32)
        return c
    lax.fori_loop(0, rows, row, 0)


def _conv_body(with_counts, x_hbm, src_hbm, dst_hbm, *rest):
    if with_counts:
        (agg_out, cnt_out, src_v, dst_v, buf, ones_v, cz_v,
         agg_sh, cnt_sh, sem) = rest
    else:
        agg_out, src_v, dst_v, buf, agg_sh, sem = rest
    cid = lax.axis_index("c")
    sid = lax.axis_index("s")

    _zero_fill_2d(buf, CH)
    for b in range(STRIPE // CH):
        pltpu.sync_copy(buf, agg_sh.at[pl.ds(sid * STRIPE + b * CH, CH)])
    if with_counts:
        for i in range(D // 16):
            ones_v[pl.ds(i * 16, 16)] = jnp.ones((16,), jnp.float32)
        def zc(i, c):
            cz_v[pl.ds(i * 16, 16)] = jnp.zeros((16,), jnp.float32)
            return c
        lax.fori_loop(0, STRIPE // 16, zc, 0)
        pltpu.sync_copy(cz_v, cnt_sh.at[pl.ds(sid * STRIPE, STRIPE)])
    plsc.subcore_barrier()

    def group(g, c):
        pltpu.sync_copy(src_hbm.at[cid, sid, pl.ds(g * G_CONV, G_CONV)], src_v)
        pltpu.sync_copy(dst_hbm.at[cid, sid, pl.ds(g * G_CONV, G_CONV)], dst_v)

        def chunk(j, c2):
            pltpu.async_copy(x_hbm.at[src_v.at[j]], buf, sem).wait()
            pltpu.sync_copy(buf, agg_sh.at[dst_v.at[j]], add=True)
            if with_counts:
                pltpu.sync_copy(ones_v, cnt_sh.at[dst_v.at[j]], add=True)
            return c2
        lax.fori_loop(0, G_CONV, chunk, 0)
        return c
    lax.fori_loop(0, K_CONV // G_CONV, group, 0)
    plsc.subcore_barrier()

    base = (1 - cid) * NPAD + sid * STRIPE
    pltpu.sync_copy(agg_sh.at[pl.ds(sid * STRIPE, STRIPE)],
                    agg_out.at[pl.ds(base, STRIPE)])
    if with_counts:
        pltpu.sync_copy(cnt_sh.at[pl.ds(sid * STRIPE, STRIPE)],
                        cnt_out.at[pl.ds(base, STRIPE)])


def _make_conv(with_counts):
    mesh = plsc.VectorSubcoreMesh(core_axis_name="c", subcore_axis_name="s")
    out_type = [jax.ShapeDtypeStruct((2 * NPAD, D), jnp.float32)]
    scratch = [
        pltpu.VMEM((G_CONV, CH), jnp.int32),
        pltpu.VMEM((G_CONV, CH), jnp.int32),
        pltpu.VMEM((CH, D), jnp.float32),
    ]
    if with_counts:
        out_type.append(jax.ShapeDtypeStruct((2 * NPAD,), jnp.float32))
        scratch += [
            pltpu.VMEM((CH,), jnp.float32),
            pltpu.VMEM((STRIPE,), jnp.float32),
        ]
    scratch.append(pltpu.VMEM_SHARED((NPAD, D), jnp.float32))
    if with_counts:
        scratch.append(pltpu.VMEM_SHARED((NPAD,), jnp.float32))
    scratch.append(pltpu.SemaphoreType.DMA)
    return pl.kernel(
        functools.partial(_conv_body, with_counts),
        out_type=tuple(out_type),
        mesh=mesh,
        scratch_types=tuple(scratch),
    )


def _decoder_body(u_hbm, row_hbm, col_hbm, v_out, row_v, col_v, buf, sem):
    cid = lax.axis_index("c")
    sid = lax.axis_index("s")
    w = cid * NTILE + sid
    pltpu.sync_copy(row_hbm.at[cid, sid], row_v)
    pltpu.sync_copy(col_hbm.at[cid, sid], col_v)

    def chunk(j, c):
        pltpu.async_copy(u_hbm.at[row_v.at[j]], buf, sem).wait()
        pltpu.async_copy(u_hbm.at[col_v.at[j]], buf, sem, add=True).wait()
        pltpu.sync_copy(buf, v_out.at[pl.ds(w * (K_LBL * CH) + j * CH, CH)])
        return c
    lax.fori_loop(0, K_LBL, chunk, 0)


_decoder_sc = pl.kernel(
    _decoder_body,
    out_type=jax.ShapeDtypeStruct((VLBL, D), jnp.float32),
    mesh=plsc.VectorSubcoreMesh(core_axis_name="c", subcore_axis_name="s"),
    scratch_types=(
        pltpu.VMEM((K_LBL, CH), jnp.int32),
        pltpu.VMEM((K_LBL, CH), jnp.int32),
        pltpu.VMEM((CH, D), jnp.float32),
        pltpu.SemaphoreType.DMA,
    ),
)


_BLK = 1024


def _layer1_tc_body(agg_ref, cnt_ref, x_ref, wl_ref, bl_ref, wr_ref, o_ref):
    inv = 1.0 / jnp.maximum(cnt_ref[...], 1.0)
    mean = agg_ref[...] * inv
    h = (jnp.dot(mean, wl_ref[0], preferred_element_type=jnp.float32)
         + bl_ref[0]
         + jnp.dot(x_ref[...], wr_ref[0], preferred_element_type=jnp.float32))
    o_ref[...] = jnp.maximum(h, 0.0)


def _layer2_tc_body(agg_ref, cnt_ref, x_ref, wl_ref, bl_ref, wr_ref,
                    w1_ref, b1_ref, o_ref):
    inv = 1.0 / jnp.maximum(cnt_ref[...], 1.0)
    mean = agg_ref[...] * inv
    z = (jnp.dot(mean, wl_ref[0], preferred_element_type=jnp.float32)
         + bl_ref[0]
         + jnp.dot(x_ref[...], wr_ref[0], preferred_element_type=jnp.float32))
    o_ref[...] = jnp.dot(z, w1_ref[0], preferred_element_type=jnp.float32) + b1_ref[0]


def _row_blk(b):
    return (b, 0)


def _w_blk(b):
    return (b // (NPAD // _BLK), 0, 0)


_N_ROWS = 2 * NPAD
_node_specs = [
    pl.BlockSpec((_BLK, D), _row_blk),
    pl.BlockSpec((_BLK, 1), _row_blk),
    pl.BlockSpec((_BLK, D), _row_blk),
    pl.BlockSpec((1, D, D), _w_blk),
    pl.BlockSpec((1, 1, D), _w_blk),
    pl.BlockSpec((1, D, D), _w_blk),
]

_layer1_tc = pl.pallas_call(
    _layer1_tc_body,
    grid=(_N_ROWS // _BLK,),
    in_specs=_node_specs,
    out_specs=pl.BlockSpec((_BLK, D), _row_blk),
    out_shape=jax.ShapeDtypeStruct((_N_ROWS, D), jnp.float32),
)

_layer2_tc = pl.pallas_call(
    _layer2_tc_body,
    grid=(_N_ROWS // _BLK,),
    in_specs=_node_specs + [
        pl.BlockSpec((1, D, D), _w_blk),
        pl.BlockSpec((1, 1, D), _w_blk),
    ],
    out_specs=pl.BlockSpec((_BLK, D), _row_blk),
    out_shape=jax.ShapeDtypeStruct((_N_ROWS, D), jnp.float32),
)


def _final_tc_body(v_ref, w2_ref, b2_ref, o_ref):
    o_ref[...] = (jnp.sum(jnp.maximum(v_ref[...], 0.0) * w2_ref[...],
                          axis=1, keepdims=True) + b2_ref[0, 0])


_FBLK = 2048
_final_tc = pl.pallas_call(
    _final_tc_body,
    grid=(VLBL // _FBLK,),
    in_specs=[
        pl.BlockSpec((_FBLK, D), _row_blk),
        pl.BlockSpec((1, D), lambda b: (0, 0)),
        pl.BlockSpec((1, 1), lambda b: (0, 0)),
    ],
    out_specs=pl.BlockSpec((_FBLK, 1), _row_blk),
    out_shape=jax.ShapeDtypeStruct((VLBL, 1), jnp.float32),
)


def _prep_edges(src, dst, src_off):
    pad = EPT - E_EDGE
    s = jnp.concatenate([src.astype(jnp.int32) + src_off,
                         jnp.full((pad,), src_off, jnp.int32)])
    d = jnp.concatenate([dst.astype(jnp.int32),
                         jnp.full((pad,), N_NODE, jnp.int32)])
    return s.reshape(NTILE, K_CONV, CH), d.reshape(NTILE, K_CONV, CH)


def kernel(x_drug, x_disease, edge_index_dd, edge_index_dr, edge_label_index,
           Wl1_dd, bl1_dd, Wr1_dd, Wl1_dr, bl1_dr, Wr1_dr,
           Wl2_dd, bl2_dd, Wr2_dd, Wl2_dr, bl2_dr, Wr2_dr,
           W1, b1, W2, b2):
    f32 = jnp.float32
    pad_n = NPAD - N_NODE
    x_cat = jnp.concatenate([
        jnp.pad(x_drug, ((0, pad_n), (0, 0))),
        jnp.pad(x_disease, ((0, pad_n), (0, 0))),
    ]).astype(f32)

    s_dd, d_dd = _prep_edges(edge_index_dd[0], edge_index_dd[1], 0)
    s_dr, d_dr = _prep_edges(edge_index_dr[0], edge_index_dr[1], NPAD)
    src_a = jnp.stack([s_dd, s_dr])
    dst_a = jnp.stack([d_dd, d_dr])

    agg1, cnt = _make_conv(True)(x_cat, src_a, dst_a)
    cnt2d = cnt.reshape(-1, 1)

    wl1 = jnp.stack([Wl1_dr, Wl1_dd])
    bl1 = jnp.stack([bl1_dr, bl1_dd]).reshape(2, 1, D)
    wr1 = jnp.stack([Wr1_dr, Wr1_dd])
    h_cat = _layer1_tc(agg1, cnt2d, x_cat, wl1, bl1, wr1)

    (agg2,) = _make_conv(False)(h_cat, src_a, dst_a)
    wl2 = jnp.stack([Wl2_dr, Wl2_dd])
    bl2 = jnp.stack([bl2_dr, bl2_dd]).reshape(2, 1, D)
    wr2 = jnp.stack([Wr2_dr, Wr2_dd])
    w1s = jnp.stack([W1[:D], W1[D:]])
    b1s = jnp.stack([b1, jnp.zeros((D,), f32)]).reshape(2, 1, D)
    u_cat = _layer2_tc(agg2, cnt2d, h_cat, wl2, bl2, wr2, w1s, b1s)

    lpad = VLBL - E_LBL
    row = jnp.concatenate([edge_label_index[0].astype(jnp.int32),
                           jnp.zeros((lpad,), jnp.int32)])
    col = jnp.concatenate([edge_label_index[1].astype(jnp.int32) + NPAD,
                           jnp.full((lpad,), NPAD, jnp.int32)])
    row_a = row.reshape(NSC, NTILE, K_LBL, CH)
    col_a = col.reshape(NSC, NTILE, K_LBL, CH)
    v = _decoder_sc(u_cat, row_a, col_a)

    out = _final_tc(v, W2.reshape(1, D), b2.reshape(1, 1))
    return out[:E_LBL, 0]


# speedup vs baseline: 4.2210x; 1.2720x over previous
import functools

import jax
import jax.numpy as jnp
from jax import lax
from jax.experimental import pallas as pl
from jax.experimental.pallas import tpu as pltpu
from jax.experimental.pallas import tpu_sc as plsc

N_NODE = 10000
NPAD = 10240
D = 128
E_EDGE = 320000
E_LBL = 100000

NSC = 2
NTILE = 16
CH = 128
K_CONV = 160
G_CONV = 16
EPT = NTILE * K_CONV * CH
STRIPE = NPAD // NTILE

VLBL = 102400
K_LBL = VLBL // (NSC * NTILE * CH)


def _zero_fill_2d(ref, rows):
    def row(r, c):
        for i in range(D // 16):
            ref[r, pl.ds(i * 16, 16)] = jnp.zeros((16,), jnp.float32)
        return c
    lax.fori_loop(0, rows, row, 0)


def _conv_body(with_counts, x_hbm, src_hbm, dst_hbm, *rest):
    if with_counts:
        (agg_out, cnt_out, srcv, dstv, buf0, buf1, ones_v, cz_v,
         agg_sh, cnt_sh, sem_g0, sem_g1, sem_is, sem_id) = rest
    else:
        (agg_out, srcv, dstv, buf0, buf1, agg_sh,
         sem_g0, sem_g1, sem_is, sem_id) = rest
    cid = lax.axis_index("c")
    sid = lax.axis_index("s")
    NG = K_CONV // G_CONV

    _zero_fill_2d(buf0, CH)
    for b in range(STRIPE // CH):
        pltpu.sync_copy(buf0, agg_sh.at[pl.ds(sid * STRIPE + b * CH, CH)])
    if with_counts:
        for i in range(D // 16):
            ones_v[pl.ds(i * 16, 16)] = jnp.ones((16,), jnp.float32)
        def zc(i, c):
            cz_v[pl.ds(i * 16, 16)] = jnp.zeros((16,), jnp.float32)
            return c
        lax.fori_loop(0, STRIPE // 16, zc, 0)
        pltpu.sync_copy(cz_v, cnt_sh.at[pl.ds(sid * STRIPE, STRIPE)])
    plsc.subcore_barrier()

    pltpu.sync_copy(src_hbm.at[cid, sid, pl.ds(0, G_CONV)], srcv.at[0])
    pltpu.sync_copy(dst_hbm.at[cid, sid, pl.ds(0, G_CONV)], dstv.at[0])
    pltpu.async_copy(x_hbm.at[srcv.at[0, 0]], buf0, sem_g0)
    pltpu.async_copy(x_hbm.at[srcv.at[0, 1]], buf1, sem_g1)

    def pair(j2, c):
        for slot, (buf, sem_g) in enumerate(((buf0, sem_g0), (buf1, sem_g1))):
            j = 2 * j2 + slot
            g = j // G_CONV
            r = j - g * G_CONV
            p = lax.rem(g, 2)

            @pl.when(jnp.logical_and(r == 0, g < NG - 1))
            def _():
                pltpu.async_copy(
                    src_hbm.at[cid, sid, pl.ds((g + 1) * G_CONV, G_CONV)],
                    srcv.at[1 - p], sem_is)
                pltpu.async_copy(
                    dst_hbm.at[cid, sid, pl.ds((g + 1) * G_CONV, G_CONV)],
                    dstv.at[1 - p], sem_id)

            pltpu.make_async_copy(x_hbm.at[srcv.at[p, r]], buf, sem_g).wait()
            pltpu.sync_copy(buf, agg_sh.at[dstv.at[p, r]], add=True)
            if with_counts:
                pltpu.sync_copy(ones_v, cnt_sh.at[dstv.at[p, r]], add=True)

            if slot == 0:
                @pl.when(jnp.logical_and(r == G_CONV - 2, g < NG - 1))
                def _():
                    pltpu.make_async_copy(
                        src_hbm.at[cid, sid, pl.ds(0, G_CONV)],
                        srcv.at[1 - p], sem_is).wait()
            else:
                @pl.when(jnp.logical_and(r == G_CONV - 1, g < NG - 1))
                def _():
                    pltpu.make_async_copy(
                        dst_hbm.at[cid, sid, pl.ds(0, G_CONV)],
                        dstv.at[1 - p], sem_id).wait()

            jn = j + 2
            gn = jn // G_CONV
            rn = jn - gn * G_CONV
            pn = lax.rem(gn, 2)

            @pl.when(jn < K_CONV)
            def _():
                pltpu.async_copy(x_hbm.at[srcv.at[pn, rn]], buf, sem_g)
        return c
    lax.fori_loop(0, K_CONV // 2, pair, 0)
    plsc.subcore_barrier()

    base = (1 - cid) * NPAD + sid * STRIPE
    pltpu.sync_copy(agg_sh.at[pl.ds(sid * STRIPE, STRIPE)],
                    agg_out.at[pl.ds(base, STRIPE)])
    if with_counts:
        pltpu.sync_copy(cnt_sh.at[pl.ds(sid * STRIPE, STRIPE)],
                        cnt_out.at[pl.ds(base, STRIPE)])


def _make_conv(with_counts):
    mesh = plsc.VectorSubcoreMesh(core_axis_name="c", subcore_axis_name="s")
    out_type = [jax.ShapeDtypeStruct((2 * NPAD, D), jnp.float32)]
    scratch = [
        pltpu.VMEM((2, G_CONV, CH), jnp.int32),
        pltpu.VMEM((2, G_CONV, CH), jnp.int32),
        pltpu.VMEM((CH, D), jnp.float32),
        pltpu.VMEM((CH, D), jnp.float32),
    ]
    if with_counts:
        out_type.append(jax.ShapeDtypeStruct((2 * NPAD,), jnp.float32))
        scratch += [
            pltpu.VMEM((CH,), jnp.float32),
            pltpu.VMEM((STRIPE,), jnp.float32),
        ]
    scratch.append(pltpu.VMEM_SHARED((NPAD, D), jnp.float32))
    if with_counts:
        scratch.append(pltpu.VMEM_SHARED((NPAD,), jnp.float32))
    scratch += [pltpu.SemaphoreType.DMA] * 4
    return pl.kernel(
        functools.partial(_conv_body, with_counts),
        out_type=tuple(out_type),
        mesh=mesh,
        scratch_types=tuple(scratch),
    )


def _decoder_body(u_hbm, row_hbm, col_hbm, v_out,
                  row_v, col_v, buf0, buf1, sem_r0, sem_r1, sem_a0, sem_a1):
    cid = lax.axis_index("c")
    sid = lax.axis_index("s")
    w = cid * NTILE + sid
    pltpu.sync_copy(row_hbm.at[cid, sid], row_v)
    pltpu.sync_copy(col_hbm.at[cid, sid], col_v)
    pltpu.async_copy(u_hbm.at[row_v.at[0]], buf0, sem_r0)
    pltpu.async_copy(u_hbm.at[row_v.at[1]], buf1, sem_r1)

    def chunk(j, buf, sem_r, sem_a):
        pltpu.make_async_copy(u_hbm.at[row_v.at[j]], buf, sem_r).wait()
        pltpu.async_copy(u_hbm.at[col_v.at[j]], buf, sem_a, add=True).wait()
        pltpu.sync_copy(buf, v_out.at[pl.ds(w * (K_LBL * CH) + j * CH, CH)])
        jn = j + 2

        @pl.when(jn < K_LBL)
        def _():
            pltpu.async_copy(u_hbm.at[row_v.at[jn]], buf, sem_r)

    def pair(j2, c):
        chunk(2 * j2, buf0, sem_r0, sem_a0)
        chunk(2 * j2 + 1, buf1, sem_r1, sem_a1)
        return c
    lax.fori_loop(0, K_LBL // 2, pair, 0)
    if K_LBL % 2:
        chunk(K_LBL - 1, buf0, sem_r0, sem_a0)


_decoder_sc = pl.kernel(
    _decoder_body,
    out_type=jax.ShapeDtypeStruct((VLBL, D), jnp.float32),
    mesh=plsc.VectorSubcoreMesh(core_axis_name="c", subcore_axis_name="s"),
    scratch_types=(
        pltpu.VMEM((K_LBL, CH), jnp.int32),
        pltpu.VMEM((K_LBL, CH), jnp.int32),
        pltpu.VMEM((CH, D), jnp.float32),
        pltpu.VMEM((CH, D), jnp.float32),
        pltpu.SemaphoreType.DMA,
        pltpu.SemaphoreType.DMA,
        pltpu.SemaphoreType.DMA,
        pltpu.SemaphoreType.DMA,
    ),
)


_BLK = 1024


def _layer1_tc_body(agg_ref, cnt_ref, x_ref, wl_ref, bl_ref, wr_ref, o_ref):
    inv = 1.0 / jnp.maximum(cnt_ref[...], 1.0)
    mean = agg_ref[...] * inv
    h = (jnp.dot(mean, wl_ref[0], preferred_element_type=jnp.float32)
         + bl_ref[0]
         + jnp.dot(x_ref[...], wr_ref[0], preferred_element_type=jnp.float32))
    o_ref[...] = jnp.maximum(h, 0.0)


def _layer2_tc_body(agg_ref, cnt_ref, x_ref, wl_ref, bl_ref, wr_ref,
                    w1_ref, b1_ref, o_ref):
    inv = 1.0 / jnp.maximum(cnt_ref[...], 1.0)
    mean = agg_ref[...] * inv
    z = (jnp.dot(mean, wl_ref[0], preferred_element_type=jnp.float32)
         + bl_ref[0]
         + jnp.dot(x_ref[...], wr_ref[0], preferred_element_type=jnp.float32))
    o_ref[...] = jnp.dot(z, w1_ref[0], preferred_element_type=jnp.float32) + b1_ref[0]


def _row_blk(b):
    return (b, 0)


def _w_blk(b):
    return (b // (NPAD // _BLK), 0, 0)


_N_ROWS = 2 * NPAD
_node_specs = [
    pl.BlockSpec((_BLK, D), _row_blk),
    pl.BlockSpec((_BLK, 1), _row_blk),
    pl.BlockSpec((_BLK, D), _row_blk),
    pl.BlockSpec((1, D, D), _w_blk),
    pl.BlockSpec((1, 1, D), _w_blk),
    pl.BlockSpec((1, D, D), _w_blk),
]

_layer1_tc = pl.pallas_call(
    _layer1_tc_body,
    grid=(_N_ROWS // _BLK,),
    in_specs=_node_specs,
    out_specs=pl.BlockSpec((_BLK, D), _row_blk),
    out_shape=jax.ShapeDtypeStruct((_N_ROWS, D), jnp.float32),
)

_layer2_tc = pl.pallas_call(
    _layer2_tc_body,
    grid=(_N_ROWS // _BLK,),
    in_specs=_node_specs + [
        pl.BlockSpec((1, D, D), _w_blk),
        pl.BlockSpec((1, 1, D), _w_blk),
    ],
    out_specs=pl.BlockSpec((_BLK, D), _row_blk),
    out_shape=jax.ShapeDtypeStruct((_N_ROWS, D), jnp.float32),
)


def _final_tc_body(v_ref, w2_ref, b2_ref, o_ref):
    o_ref[...] = (jnp.sum(jnp.maximum(v_ref[...], 0.0) * w2_ref[...],
                          axis=1, keepdims=True) + b2_ref[0, 0])


_FBLK = 2048
_final_tc = pl.pallas_call(
    _final_tc_body,
    grid=(VLBL // _FBLK,),
    in_specs=[
        pl.BlockSpec((_FBLK, D), _row_blk),
        pl.BlockSpec((1, D), lambda b: (0, 0)),
        pl.BlockSpec((1, 1), lambda b: (0, 0)),
    ],
    out_specs=pl.BlockSpec((_FBLK, 1), _row_blk),
    out_shape=jax.ShapeDtypeStruct((VLBL, 1), jnp.float32),
)


def _prep_edges(src, dst, src_off):
    pad = EPT - E_EDGE
    s = jnp.concatenate([src.astype(jnp.int32) + src_off,
                         jnp.full((pad,), src_off, jnp.int32)])
    d = jnp.concatenate([dst.astype(jnp.int32),
                         jnp.full((pad,), N_NODE, jnp.int32)])
    return s.reshape(NTILE, K_CONV, CH), d.reshape(NTILE, K_CONV, CH)


def kernel(x_drug, x_disease, edge_index_dd, edge_index_dr, edge_label_index,
           Wl1_dd, bl1_dd, Wr1_dd, Wl1_dr, bl1_dr, Wr1_dr,
           Wl2_dd, bl2_dd, Wr2_dd, Wl2_dr, bl2_dr, Wr2_dr,
           W1, b1, W2, b2):
    f32 = jnp.float32
    pad_n = NPAD - N_NODE
    x_cat = jnp.concatenate([
        jnp.pad(x_drug, ((0, pad_n), (0, 0))),
        jnp.pad(x_disease, ((0, pad_n), (0, 0))),
    ]).astype(f32)

    s_dd, d_dd = _prep_edges(edge_index_dd[0], edge_index_dd[1], 0)
    s_dr, d_dr = _prep_edges(edge_index_dr[0], edge_index_dr[1], NPAD)
    src_a = jnp.stack([s_dd, s_dr])
    dst_a = jnp.stack([d_dd, d_dr])

    agg1, cnt = _make_conv(True)(x_cat, src_a, dst_a)
    cnt2d = cnt.reshape(-1, 1)

    wl1 = jnp.stack([Wl1_dr, Wl1_dd])
    bl1 = jnp.stack([bl1_dr, bl1_dd]).reshape(2, 1, D)
    wr1 = jnp.stack([Wr1_dr, Wr1_dd])
    h_cat = _layer1_tc(agg1, cnt2d, x_cat, wl1, bl1, wr1)

    (agg2,) = _make_conv(False)(h_cat, src_a, dst_a)
    wl2 = jnp.stack([Wl2_dr, Wl2_dd])
    bl2 = jnp.stack([bl2_dr, bl2_dd]).reshape(2, 1, D)
    wr2 = jnp.stack([Wr2_dr, Wr2_dd])
    w1s = jnp.stack([W1[:D], W1[D:]])
    b1s = jnp.stack([b1, jnp.zeros((D,), f32)]).reshape(2, 1, D)
    u_cat = _layer2_tc(agg2, cnt2d, h_cat, wl2, bl2, wr2, w1s, b1s)

    lpad = VLBL - E_LBL
    row = jnp.concatenate([edge_label_index[0].astype(jnp.int32),
                           jnp.zeros((lpad,), jnp.int32)])
    col = jnp.concatenate([edge_label_index[1].astype(jnp.int32) + NPAD,
                           jnp.full((lpad,), NPAD, jnp.int32)])
    row_a = row.reshape(NSC, NTILE, K_LBL, CH)
    col_a = col.reshape(NSC, NTILE, K_LBL, CH)
    v = _decoder_sc(u_cat, row_a, col_a)

    out = _final_tc(v, W2.reshape(1, D), b2.reshape(1, 1))
    return out[:E_LBL, 0]
